# baseline (speedup 1.0000x reference)
import jax
import jax.numpy as jnp
from jax import lax
from jax.experimental import pallas as pl

_K = 64


def _readout_kernel(x_ref, o_ref):
    x = x_ref[0]
    n, d = x.shape
    rows = 8
    cols = n // rows
    x3 = x.reshape(rows, cols, d)
    s = jnp.sqrt(jnp.sum(x3 * x3, axis=2))
    sb = lax.bitcast_convert_type(s, jnp.int32)

    def tbody(i, t):
        cand = t | lax.shift_left(jnp.int32(1), 30 - i)
        cnt = jnp.sum((sb >= cand).astype(jnp.int32))
        return jnp.where(cnt >= _K, cand, t)

    t = lax.fori_loop(0, 31, tbody, jnp.int32(0))

    gt = sb > t
    m = jnp.sum(gt.astype(jnp.int32))
    r = _K - m
    tie = sb == t
    row_i = lax.broadcasted_iota(jnp.int32, (rows, cols), 0)
    col_i = lax.broadcasted_iota(jnp.int32, (rows, cols), 1)
    idxp = (n - 1) - (row_i * cols + col_i)

    def ubody(i, u):
        cand = u | lax.shift_left(jnp.int32(1), 11 - i)
        cnt = jnp.sum((tie & (idxp >= cand)).astype(jnp.int32))
        return jnp.where(cnt >= r, cand, u)

    u = lax.fori_loop(0, 12, ubody, jnp.int32(0))

    mask = gt | (tie & (idxp >= u))
    mf = mask.astype(jnp.float32).reshape(1, n)
    out = lax.dot_general(mf, x, (((1,), (0,)), ((), ())),
                          preferred_element_type=jnp.float32)
    o_ref[0] = out * (1.0 / _K)


def kernel(H_prime):
    b, n, d = H_prime.shape
    out = pl.pallas_call(
        _readout_kernel,
        grid=(b,),
        in_specs=[pl.BlockSpec((1, n, d), lambda i: (i, 0, 0))],
        out_specs=pl.BlockSpec((1, 1, d), lambda i: (i, 0, 0)),
        out_shape=jax.ShapeDtypeStruct((b, 1, d), jnp.float32),
    )(H_prime)
    return out.reshape(b, d)


# speedup vs baseline: 5.9295x; 5.9295x over previous
import functools

import jax
import jax.numpy as jnp
from jax import lax
from jax.experimental import pallas as pl
from jax.experimental.pallas import tpu as pltpu
from jax.experimental.pallas import tpu_sc as plsc

_K = 64


def _scores_kernel(x_ref, idx_ref, scores_ref):
    nb = pl.num_programs(0)
    b = pl.program_id(0)
    x = x_ref[0]
    n, _ = x.shape
    s = jnp.sqrt(jnp.sum(x * x, axis=1))
    scores_ref[pl.ds(b, 1), :] = s.reshape(1, n)

    @pl.when(b == nb - 1)
    def _select():
        s_all = scores_ref[...]
        nb_ = s_all.shape[0]
        sb = lax.bitcast_convert_type(s_all, jnp.int32)

        def tbody(i, t):
            cand = t | lax.shift_left(jnp.int32(1), 30 - i)
            cnt = jnp.sum((sb >= cand).astype(jnp.int32), axis=1, keepdims=True)
            return jnp.where(cnt >= _K, cand, t)

        t = lax.fori_loop(0, 31, tbody, jnp.zeros((nb_, 1), jnp.int32))

        gt = sb > t
        m = jnp.sum(gt.astype(jnp.int32), axis=1, keepdims=True)
        r = _K - m
        tie = sb == t
        idxp = (n - 1) - lax.broadcasted_iota(jnp.int32, sb.shape, 1)

        def ubody(i, u):
            cand = u | lax.shift_left(jnp.int32(1), 11 - i)
            cnt = jnp.sum((tie & (idxp >= cand)).astype(jnp.int32),
                          axis=1, keepdims=True)
            return jnp.where(cnt >= r, cand, u)

        u = lax.fori_loop(0, 12, ubody, jnp.zeros((nb_, 1), jnp.int32))

        msk = (gt | (tie & (idxp >= u))).astype(jnp.int32)

        p = msk
        sh = 1
        while sh < n:
            p = p + jnp.concatenate(
                [jnp.zeros((nb_, sh), jnp.int32), p[:, :n - sh]], axis=1)
            sh *= 2

        lane64 = lax.broadcasted_iota(jnp.int32, (nb_, _K), 1)

        def jbody(j, acc):
            cnt = jnp.sum((p <= j).astype(jnp.int32), axis=1, keepdims=True)
            return acc + cnt * (lane64 == j).astype(jnp.int32)

        idx_mat = lax.fori_loop(0, _K, jbody,
                                jnp.zeros((nb_, _K), jnp.int32))
        base = lax.broadcasted_iota(jnp.int32, (nb_, _K), 0) * n
        idx_ref[...] = idx_mat + base


def _make_gather_mean(b_, n, d):
    mesh = plsc.VectorSubcoreMesh(core_axis_name="c", subcore_axis_name="s")
    d_chunks = d // 16

    @functools.partial(
        pl.kernel,
        mesh=mesh,
        out_type=jax.ShapeDtypeStruct((b_, d), jnp.float32),
        scratch_types=[
            pltpu.VMEM((_K,), jnp.int32),
            pltpu.VMEM((_K, d), jnp.float32),
            pltpu.VMEM((1, d), jnp.float32),
            pltpu.SemaphoreType.DMA,
        ],
    )
    def gather_mean(idx_hbm, tbl_hbm, out_hbm, idx_v, rows_v, o_v, sem):
        wid = lax.axis_index("s") * 2 + lax.axis_index("c")

        @pl.when(wid < b_)
        def _():
            pltpu.sync_copy(idx_hbm.at[pl.ds(wid * _K, _K)], idx_v)
            pltpu.async_copy(tbl_hbm.at[idx_v], rows_v, sem).wait()

            zeros = [jnp.zeros((16,), jnp.float32) for _ in range(d_chunks)]

            def rbody(rj, accs):
                return tuple(accs[c2] + rows_v[rj, pl.ds(c2 * 16, 16)]
                             for c2 in range(d_chunks))

            accs = lax.fori_loop(0, _K, rbody, tuple(zeros))
            for c2 in range(d_chunks):
                o_v[0, pl.ds(c2 * 16, 16)] = accs[c2] * (1.0 / _K)
            pltpu.sync_copy(o_v, out_hbm.at[pl.ds(wid, 1)])

    return gather_mean


def kernel(H_prime):
    b, n, d = H_prime.shape
    idx = pl.pallas_call(
        _scores_kernel,
        grid=(b,),
        in_specs=[pl.BlockSpec((1, n, d), lambda i: (i, 0, 0))],
        out_specs=pl.BlockSpec((b, _K), lambda i: (0, 0)),
        out_shape=jax.ShapeDtypeStruct((b, _K), jnp.int32),
        scratch_shapes=[pltpu.VMEM((b, n), jnp.float32)],
    )(H_prime)
    tbl = H_prime.reshape(b * n, d)
    return _make_gather_mean(b, n, d)(idx.reshape(b * _K), tbl)
